# baseline (speedup 1.0000x reference)
import functools

import jax
import jax.numpy as jnp
from jax import lax
from jax.experimental import pallas as pl
from jax.experimental.pallas import tpu as pltpu
from jax.experimental.pallas import tpu_sc as plsc

_D_MODEL = 2048
_N_EXPERTS = 64
_N_TOKENS = 16384
_TB = 2048
_NW = 32
_TPW = _N_TOKENS // _NW


def _logits_t_body(x_ref, wt_ref, lt_ref):
    logits = jnp.dot(x_ref[...], wt_ref[...], preferred_element_type=jnp.float32)
    lt_ref[...] = logits.T


def _sc_topk(lt_hbm, wt_out, et_out, lchunk, w1b, w2b, i1b, i2b):
    wid = lax.axis_index("s") * 2 + lax.axis_index("c")
    base = wid * _TPW
    pltpu.sync_copy(lt_hbm.at[:, pl.ds(base, _TPW)], lchunk)

    def group(g, carry):
        col = g * 16
        m1 = jnp.full((16,), -jnp.inf, jnp.float32)
        m2 = jnp.full((16,), -jnp.inf, jnp.float32)
        i1 = jnp.zeros((16,), jnp.int32)
        i2 = jnp.zeros((16,), jnp.int32)
        for e in range(_N_EXPERTS):
            v = lchunk[e, pl.ds(col, 16)]
            beats1 = v > m1
            beats2 = v > m2
            m2 = jnp.where(beats1, m1, jnp.where(beats2, v, m2))
            i2 = jnp.where(beats1, i1, jnp.where(beats2, e, i2))
            m1 = jnp.where(beats1, v, m1)
            i1 = jnp.where(beats1, e, i1)
        w1 = 1.0 / (1.0 + jnp.exp(m2 - m1))
        w1b[pl.ds(col, 16)] = w1
        w2b[pl.ds(col, 16)] = 1.0 - w1
        i1b[pl.ds(col, 16)] = i1
        i2b[pl.ds(col, 16)] = i2
        return carry

    lax.fori_loop(0, _TPW // 16, group, 0)
    pltpu.sync_copy(w1b, wt_out.at[0, pl.ds(base, _TPW)])
    pltpu.sync_copy(w2b, wt_out.at[1, pl.ds(base, _TPW)])
    pltpu.sync_copy(i1b, et_out.at[0, pl.ds(base, _TPW)])
    pltpu.sync_copy(i2b, et_out.at[1, pl.ds(base, _TPW)])


def kernel(x, W):
    wt = W.T
    n_tokens = x.shape[0]
    lt = pl.pallas_call(
        _logits_t_body,
        grid=(n_tokens // _TB,),
        in_specs=[
            pl.BlockSpec((_TB, _D_MODEL), lambda i: (i, 0)),
            pl.BlockSpec((_D_MODEL, _N_EXPERTS), lambda i: (0, 0)),
        ],
        out_specs=pl.BlockSpec((_N_EXPERTS, _TB), lambda i: (0, i)),
        out_shape=jax.ShapeDtypeStruct((_N_EXPERTS, n_tokens), jnp.float32),
        compiler_params=pltpu.CompilerParams(
            dimension_semantics=("parallel",),
        ),
    )(x, wt)

    sc_call = pl.kernel(
        _sc_topk,
        out_type=[
            jax.ShapeDtypeStruct((2, n_tokens), jnp.float32),
            jax.ShapeDtypeStruct((2, n_tokens), jnp.int32),
        ],
        mesh=plsc.VectorSubcoreMesh(core_axis_name="c", subcore_axis_name="s"),
        scratch_types=[
            pltpu.VMEM((_N_EXPERTS, _TPW), jnp.float32),
            pltpu.VMEM((_TPW,), jnp.float32),
            pltpu.VMEM((_TPW,), jnp.float32),
            pltpu.VMEM((_TPW,), jnp.int32),
            pltpu.VMEM((_TPW,), jnp.int32),
        ],
    )
    wT, eT = sc_call(lt)
    return (wT.T, eT.T)


# speedup vs baseline: 1.1829x; 1.1829x over previous
import jax
import jax.numpy as jnp
from jax.experimental import pallas as pl
from jax.experimental.pallas import tpu as pltpu

_D_MODEL = 2048
_N_EXPERTS = 64
_N_TOKENS = 16384
_TB = 2048


def _router_body(x_ref, w_ref, w_out_ref, e_out_ref):
    logits = jax.lax.dot_general(
        x_ref[...],
        w_ref[...],
        dimension_numbers=(((1,), (1,)), ((), ())),
        preferred_element_type=jnp.float32,
    )
    iota = jax.lax.broadcasted_iota(jnp.int32, logits.shape, 1)

    m1 = jnp.max(logits, axis=1, keepdims=True)
    idx1 = jnp.argmax(logits, axis=1, keepdims=True)
    masked = jnp.where(iota == idx1, -jnp.inf, logits)
    m2 = jnp.max(masked, axis=1, keepdims=True)
    idx2 = jnp.argmax(masked, axis=1, keepdims=True)

    w1 = jax.nn.sigmoid(m1 - m2)
    w_out_ref[...] = jnp.concatenate([w1, 1.0 - w1], axis=1)
    e_out_ref[...] = jnp.concatenate([idx1, idx2], axis=1)


def kernel(x, W):
    n_tokens = x.shape[0]
    grid = (n_tokens // _TB,)
    weights, experts = pl.pallas_call(
        _router_body,
        grid=grid,
        in_specs=[
            pl.BlockSpec((_TB, _D_MODEL), lambda i: (i, 0)),
            pl.BlockSpec((_N_EXPERTS, _D_MODEL), lambda i: (0, 0)),
        ],
        out_specs=[
            pl.BlockSpec((_TB, 2), lambda i: (i, 0)),
            pl.BlockSpec((_TB, 2), lambda i: (i, 0)),
        ],
        out_shape=[
            jax.ShapeDtypeStruct((n_tokens, 2), jnp.float32),
            jax.ShapeDtypeStruct((n_tokens, 2), jnp.int32),
        ],
        compiler_params=pltpu.CompilerParams(
            dimension_semantics=("parallel",),
        ),
    )(x, W)
    return (weights, experts)
